# Initial kernel scaffold; baseline (speedup 1.0000x reference)
import jax
import jax.numpy as jnp
from jax.experimental import pallas as pl


def kernel(x, edge_index, batch, W1, b1, W2, b2, W3, b3, Wh, bh):
    raise NotImplementedError("write your pallas kernel here")


# speedup vs baseline: 18.6152x; 18.6152x over previous
import functools

import jax
import jax.numpy as jnp
from jax import lax
from jax.experimental import pallas as pl
from jax.experimental.pallas import tpu as pltpu
from jax.experimental.pallas import tpu_sc as plsc

N = 10000
E = 320000
G = 64
D = 128
NCLS = 2

NC = 2
NS = 16
NW = NC * NS
EPW = E // NW
C = 125
NCH = EPW // C
NR = 10240
STRIPE = NR // NS
ZROWS = 128
NA = 10240
SA = NA // NS


def _sc_mesh():
    return plsc.VectorSubcoreMesh(
        core_axis_name="c", subcore_axis_name="s",
        num_cores=NC, num_subcores=NS)


@functools.cache
def _deg_kernel_build():
    return functools.partial(
        pl.kernel,
        out_type=jax.ShapeDtypeStruct((NC, NA), jnp.float32),
        mesh=_sc_mesh(),
        scratch_types=[
            pltpu.VMEM((NCH, C), jnp.int32),
            pltpu.VMEM((128,), jnp.float32),
            pltpu.VMEM_SHARED((NA,), jnp.float32),
        ],
    )(_deg_body)


def _deg_body(dst_hbm, ones_hbm, z1_hbm, out_hbm, dst_v, ones_v, acc_sh):
    c = lax.axis_index("c")
    s = lax.axis_index("s")
    w = c * NS + s

    pltpu.sync_copy(z1_hbm.at[pl.ds(s * SA, SA)],
                    acc_sh.at[pl.ds(s * SA, SA)])
    pltpu.sync_copy(ones_hbm, ones_v)
    pltpu.sync_copy(dst_hbm.at[w], dst_v)
    plsc.subcore_barrier()

    @pl.loop(0, NCH)
    def _chunks(j):
        pltpu.sync_copy(ones_v.at[pl.ds(0, C)], acc_sh.at[dst_v.at[j]],
                        add=True)

    plsc.subcore_barrier()
    pltpu.sync_copy(acc_sh.at[pl.ds(s * SA, SA)],
                    out_hbm.at[c, pl.ds(s * SA, SA)])


@functools.cache
def _agg_kernel_build():
    return functools.partial(
        pl.kernel,
        out_type=jax.ShapeDtypeStruct((NC, NR, D), jnp.float32),
        mesh=_sc_mesh(),
        scratch_types=[
            pltpu.VMEM((NCH, C), jnp.int32),
            pltpu.VMEM((NCH, C), jnp.int32),
            pltpu.VMEM((C, D), jnp.float32),
            pltpu.VMEM_SHARED((NR, D), jnp.float32),
        ],
    )(_agg_body)


def _agg_body(g_hbm, src_hbm, dst_hbm, z_hbm, out_hbm, src_v, dst_v, rows_v,
              acc_sh):
    c = lax.axis_index("c")
    s = lax.axis_index("s")
    w = c * NS + s

    pltpu.sync_copy(z_hbm.at[pl.ds(s * STRIPE, STRIPE)],
                    acc_sh.at[pl.ds(s * STRIPE, STRIPE)])
    pltpu.sync_copy(src_hbm.at[w], src_v)
    pltpu.sync_copy(dst_hbm.at[w], dst_v)
    plsc.subcore_barrier()

    @pl.loop(0, NCH)
    def _chunks(j):
        pltpu.sync_copy(g_hbm.at[src_v.at[j]], rows_v)
        pltpu.sync_copy(rows_v, acc_sh.at[dst_v.at[j]], add=True)

    plsc.subcore_barrier()
    pltpu.sync_copy(acc_sh.at[pl.ds(s * STRIPE, STRIPE)],
                    out_hbm.at[c, pl.ds(s * STRIPE, STRIPE)])


BR = 1000


def _prep_body(p0, p1, x, w1, dinv_ref, g_ref):
    deg = p0[...] + p1[...] + 1.0
    dinv = lax.rsqrt(deg)
    dinv_ref[...] = dinv
    g_ref[...] = dinv * jnp.dot(x[...], w1[...],
                                preferred_element_type=jnp.float32)


def _prep(p0, p1, x, w1):
    return pl.pallas_call(
        _prep_body,
        grid=(N // BR,),
        in_specs=[
            pl.BlockSpec((BR, 1), lambda i: (i, 0)),
            pl.BlockSpec((BR, 1), lambda i: (i, 0)),
            pl.BlockSpec((BR, D), lambda i: (i, 0)),
            pl.BlockSpec((D, D), lambda i: (0, 0)),
        ],
        out_specs=[
            pl.BlockSpec((BR, 1), lambda i: (i, 0)),
            pl.BlockSpec((BR, D), lambda i: (i, 0)),
        ],
        out_shape=[
            jax.ShapeDtypeStruct((N, 1), jnp.float32),
            jax.ShapeDtypeStruct((N, D), jnp.float32),
        ],
    )(p0, p1, x, w1)


def _layer_body(a0, a1, g, dinv, b, w, gn_ref):
    h = jnp.maximum(dinv[...] * (a0[...] + a1[...] + g[...]) + b[...], 0.0)
    gn_ref[...] = dinv[...] * jnp.dot(h, w[...],
                                      preferred_element_type=jnp.float32)


def _layer(a0, a1, g, dinv, b, w):
    return pl.pallas_call(
        _layer_body,
        grid=(N // BR,),
        in_specs=[
            pl.BlockSpec((BR, D), lambda i: (i, 0)),
            pl.BlockSpec((BR, D), lambda i: (i, 0)),
            pl.BlockSpec((BR, D), lambda i: (i, 0)),
            pl.BlockSpec((BR, 1), lambda i: (i, 0)),
            pl.BlockSpec((1, D), lambda i: (0, 0)),
            pl.BlockSpec((D, D), lambda i: (0, 0)),
        ],
        out_specs=pl.BlockSpec((BR, D), lambda i: (i, 0)),
        out_shape=jax.ShapeDtypeStruct((N, D), jnp.float32),
    )(a0, a1, g, dinv, b, w)


def _head_body(a0, a1, g, dinv, b, batch, wh, bh, out_ref, psum, cnt):
    i = pl.program_id(0)

    @pl.when(i == 0)
    def _():
        psum[...] = jnp.zeros_like(psum)
        cnt[...] = jnp.zeros_like(cnt)

    h = jnp.maximum(dinv[...] * (a0[...] + a1[...] + g[...]) + b[...], 0.0)
    sel = (batch[...] == lax.broadcasted_iota(jnp.int32, (BR, G), 1)
           ).astype(jnp.float32)
    dn = (((0,), (0,)), ((), ()))
    psum[...] += lax.dot_general(sel, h, dn,
                                 preferred_element_type=jnp.float32)
    cnt[...] += lax.dot_general(sel, jnp.ones((BR, 1), jnp.float32), dn,
                                preferred_element_type=jnp.float32)

    @pl.when(i == pl.num_programs(0) - 1)
    def _():
        pooled = psum[...] / jnp.maximum(cnt[...], 1.0)
        out_ref[...] = jnp.dot(pooled, wh[...],
                               preferred_element_type=jnp.float32) + bh[...]


def _head(a0, a1, g, dinv, b, batch, wh, bh):
    return pl.pallas_call(
        _head_body,
        grid=(N // BR,),
        in_specs=[
            pl.BlockSpec((BR, D), lambda i: (i, 0)),
            pl.BlockSpec((BR, D), lambda i: (i, 0)),
            pl.BlockSpec((BR, D), lambda i: (i, 0)),
            pl.BlockSpec((BR, 1), lambda i: (i, 0)),
            pl.BlockSpec((1, D), lambda i: (0, 0)),
            pl.BlockSpec((BR, 1), lambda i: (i, 0)),
            pl.BlockSpec((D, NCLS), lambda i: (0, 0)),
            pl.BlockSpec((1, NCLS), lambda i: (0, 0)),
        ],
        out_specs=pl.BlockSpec((G, NCLS), lambda i: (0, 0)),
        out_shape=jax.ShapeDtypeStruct((G, NCLS), jnp.float32),
        scratch_shapes=[
            pltpu.VMEM((G, D), jnp.float32),
            pltpu.VMEM((G, 1), jnp.float32),
        ],
    )(a0, a1, g, dinv, b, batch, wh, bh)


def kernel(x, edge_index, batch, W1, b1, W2, b2, W3, b3, Wh, bh):
    src = edge_index[0].astype(jnp.int32).reshape(NW, NCH, C)
    dst = edge_index[1].astype(jnp.int32).reshape(NW, NCH, C)

    ones128 = jnp.ones((128,), jnp.float32)
    zrow = jnp.zeros((NR, D), jnp.float32)
    z1 = jnp.zeros((NA,), jnp.float32)

    degp = _deg_kernel_build()(dst, ones128, z1)
    p0 = degp[0, :N].reshape(N, 1)
    p1 = degp[1, :N].reshape(N, 1)

    dinv, g1 = _prep(p0, p1, x, W1)
    agg = _agg_kernel_build()

    acc = agg(g1, src, dst, zrow)
    g2 = _layer(acc[0], acc[1], g1, dinv, b1.reshape(1, D), W2)
    acc = agg(g2, src, dst, zrow)
    g3 = _layer(acc[0], acc[1], g2, dinv, b2.reshape(1, D), W3)
    acc = agg(g3, src, dst, zrow)

    return _head(acc[0], acc[1], g3, dinv, b3.reshape(1, D),
                 batch.astype(jnp.int32).reshape(N, 1),
                 Wh, bh.reshape(1, NCLS))
